# baseline (device time: 211452 ns/iter reference)
import jax
import jax.numpy as jnp
from jax import lax
from jax.experimental import pallas as pl
from jax.experimental.pallas import tpu as pltpu

N_DEV = 16
B, Sq, D = 4, 256, 1024
HL, Dh = 8, 128
KVL = 2
GRP = 4
Skv = 1024
R = B * Sq
CH = R // N_DEV
SCALE = 0.08838834764831843
BF = jnp.bfloat16
F32 = jnp.float32


def _rdma(src, dst, send_sem, recv_sem, dev):
    return pltpu.make_async_remote_copy(
        src_ref=src, dst_ref=dst, send_sem=send_sem, recv_sem=recv_sem,
        device_id=(dev,), device_id_type=pl.DeviceIdType.MESH,
    )


def _body(x_ref, wq_ref, wo_ref, k_hbm, v_hbm, out_ref,
          wq16, wo16, kvm, vvm,
          st16, pst16, rs16, zA16, zB16, agstR, agstL, agR, agL,
          kv_sems,
          p1_send_sems, p1_recv_sems,
          p2_send_sem, p2_recv_sems,
          p3r_send_sems, p3r_recv_sems,
          p3l_send_sems, p3l_recv_sems):
    my = lax.axis_index("i")
    z = my // 4
    w = my % 4
    zb0 = z % 2
    zb1 = (z // 2) % 2
    wr = z * 4 + (w + 1) % 4
    wl = z * 4 + (w + 3) % 4
    pA = my + 4 - 8 * zb0
    pB = my + 8 - 16 * zb1

    ck = pltpu.make_async_copy(
        k_hbm.at[:, :, pl.ds(KVL * Dh * my, KVL * Dh)], kvm, kv_sems.at[0])
    cv = pltpu.make_async_copy(
        v_hbm.at[:, :, pl.ds(KVL * Dh * my, KVL * Dh)], vvm, kv_sems.at[1])
    ck.start()
    cv.start()
    wq16[...] = wq_ref[...].astype(BF)
    wo16[...] = wo_ref[...].astype(BF)

    barrier = pltpu.get_barrier_semaphore()
    for nbr in (wl, wr, pA, pB):
        pl.semaphore_signal(barrier, inc=1, device_id=(nbr,),
                            device_id_type=pl.DeviceIdType.MESH)
    pl.semaphore_wait(barrier, 4)
    ck.wait()
    cv.wait()

    def compute_group(g):
        xg = x_ref[pl.ds(Sq * g, Sq)].astype(BF)
        qg = jnp.dot(xg, wq16[...],
                     preferred_element_type=F32)
        kb = kvm[pl.ds(g, 1)].reshape(Skv, KVL * Dh).astype(BF)
        vb = vvm[pl.ds(g, 1)].reshape(Skv, KVL * Dh).astype(BF)
        heads = []
        for h in range(HL):
            kv = h // GRP
            qh = qg[:, h * Dh:(h + 1) * Dh].astype(BF)
            khd = kb[:, kv * Dh:(kv + 1) * Dh]
            s = lax.dot_general(
                qh, khd, (((1,), (1,)), ((), ())),
                preferred_element_type=F32) * SCALE
            m = jnp.max(s, axis=1, keepdims=True)
            p = jnp.exp(s - m)
            l = jnp.sum(p, axis=1, keepdims=True)
            o = jnp.dot(p.astype(BF), vb[:, kv * Dh:(kv + 1) * Dh],
                        preferred_element_type=F32)
            heads.append(o / l)
        rowb = jnp.concatenate(heads, axis=1).astype(BF)
        partial = jnp.dot(rowb, wo16[...],
                          preferred_element_type=F32)
        out_ref[pl.ds(4 * g, 4)] = partial.reshape(4, CH, D)

    compute_group(w)
    p1_descs = []
    for s in range(3):
        g = (w - s + 4) % 4
        st16[s] = out_ref[pl.ds(4 * g, 4)].astype(BF)
        r = _rdma(st16.at[s], rs16.at[s],
                  p1_send_sems.at[s], p1_recv_sems.at[s], wr)
        r.start()
        p1_descs.append(r)
        nxt = (w - 1 - s + 4) % 4
        compute_group(nxt)
        r.wait_recv()
        out_ref[pl.ds(4 * nxt, 4)] = (out_ref[pl.ds(4 * nxt, 4)]
                                      + rs16[s].astype(F32))
    for r in p1_descs:
        r.wait_send()
    G = (w + 1) % 4
    base = 4 * G

    pst16[...] = out_ref[pl.ds(base, 4)].astype(BF)
    r = _rdma(pst16, zA16, p2_send_sem, p2_recv_sems.at[0], pA)
    r.start()
    r.wait()
    out_ref[pl.ds(base, 4)] = (out_ref[pl.ds(base, 4)]
                               + zA16[...].astype(F32))
    pst16[...] = out_ref[pl.ds(base, 4)].astype(BF)
    r = _rdma(pst16, zB16, p2_send_sem, p2_recv_sems.at[1], pB)
    r.start()
    r.wait()
    out_ref[pl.ds(base, 4)] = (out_ref[pl.ds(base, 4)]
                               + zB16[...].astype(F32))

    agstR[...] = out_ref[pl.ds(base, 2)].astype(BF)
    agstL[...] = out_ref[pl.ds(base + 2, 2)].astype(BF)
    p3_descs = []
    for s in range(3):
        srcR = agstR if s == 0 else agR.at[s - 1]
        rR = _rdma(srcR, agR.at[s],
                   p3r_send_sems.at[s], p3r_recv_sems.at[s], wr)
        rR.start()
        srcL = agstL if s == 0 else agL.at[s - 1]
        rL = _rdma(srcL, agL.at[s],
                   p3l_send_sems.at[s], p3l_recv_sems.at[s], wl)
        rL.start()
        p3_descs += [rR, rL]
        if s > 0:
            gr = (w - (s - 1) + 4) % 4
            out_ref[pl.ds(4 * gr, 2)] = agR[s - 1].astype(F32)
            gl = (w + 2 + (s - 1)) % 4
            out_ref[pl.ds(4 * gl + 2, 2)] = agL[s - 1].astype(F32)
        rR.wait_recv()
        rL.wait_recv()
    out_ref[pl.ds(4 * ((w - 2 + 4) % 4), 2)] = agR[2].astype(F32)
    out_ref[pl.ds(4 * (w % 4) + 2, 2)] = agL[2].astype(F32)
    for r in p3_descs:
        r.wait_send()


def kernel(x, Wq, Wo, K_ext, V_ext):
    x2 = x.reshape(R, D)
    K2 = K_ext.reshape(B, Skv, 32 * Dh)
    V2 = V_ext.reshape(B, Skv, 32 * Dh)

    out = pl.pallas_call(
        _body,
        out_shape=jax.ShapeDtypeStruct((N_DEV, CH, D), jnp.float32),
        in_specs=[pl.BlockSpec(memory_space=pltpu.VMEM)] * 3
        + [pl.BlockSpec(memory_space=pltpu.MemorySpace.HBM)] * 2,
        out_specs=pl.BlockSpec(memory_space=pltpu.VMEM),
        scratch_shapes=[
            pltpu.VMEM((D, D), BF),
            pltpu.VMEM((D, D), BF),
            pltpu.VMEM((B, Skv, KVL * Dh), F32),
            pltpu.VMEM((B, Skv, KVL * Dh), F32),
            pltpu.VMEM((3, 4, CH, D), BF),
            pltpu.VMEM((4, CH, D), BF),
            pltpu.VMEM((3, 4, CH, D), BF),
            pltpu.VMEM((4, CH, D), BF),
            pltpu.VMEM((4, CH, D), BF),
            pltpu.VMEM((2, CH, D), BF),
            pltpu.VMEM((2, CH, D), BF),
            pltpu.VMEM((3, 2, CH, D), BF),
            pltpu.VMEM((3, 2, CH, D), BF),
            pltpu.SemaphoreType.DMA((2,)),
            pltpu.SemaphoreType.DMA((3,)),
            pltpu.SemaphoreType.DMA((3,)),
            pltpu.SemaphoreType.DMA,
            pltpu.SemaphoreType.DMA((2,)),
            pltpu.SemaphoreType.DMA((3,)),
            pltpu.SemaphoreType.DMA((3,)),
            pltpu.SemaphoreType.DMA((3,)),
            pltpu.SemaphoreType.DMA((3,)),
        ],
        compiler_params=pltpu.CompilerParams(collective_id=0),
    )(x2, Wq, Wo, K2, V2)
    return out.reshape(B, Sq, D)
